# baseline (speedup 1.0000x reference)
import functools

import jax
import jax.numpy as jnp
from jax import lax
from jax.experimental import pallas as pl
from jax.experimental.pallas import tpu as pltpu
from jax.experimental.pallas import tpu_sc as plsc

D = 2048
T = 2048
B = 2
N = 8192
EPS = 1e-8

NC = 2
NS = 16
NW = NC * NS
L = 16

GROUPS = N // L
GPW = GROUPS // NW
UNROLL = 8


def _sc_body(zt_hbm, r_hbm, ii_hbm, jj_hbm, num_hbm, s_hbm,
             ii_v, jj_v, ridx_v, r_v, rows_i, rows_j, o_v, s_v, sem):
  wid = lax.axis_index("s") * NC + lax.axis_index("c")
  lanes = lax.iota(jnp.int32, L)

  def group_body(gl, carry0):
    g = wid * GPW + gl
    pbase = g * L
    pltpu.sync_copy(ii_hbm.at[pl.ds(pbase, L)], ii_v)
    pltpu.sync_copy(jj_hbm.at[pl.ds(pbase, L)], jj_v)
    pltpu.sync_copy(r_hbm.at[pl.ds(pbase, L)], r_v)
    d = jnp.exp(-r_v[...])

    for b in range(B):
      ridx_v[...] = ii_v[...] + b * D
      pltpu.async_copy(zt_hbm.at[ridx_v], rows_i, sem).wait()
      ridx_v[...] = jj_v[...] + b * D
      pltpu.async_copy(zt_hbm.at[ridx_v], rows_j, sem).wait()

      def t_chunk(c, ch_carry):
        w, acc, ssum, tvec = ch_carry
        for _ in range(UNROLL):
          zi = plsc.load_gather(rows_i, [lanes, tvec])
          zj = plsc.load_gather(rows_j, [lanes, tvec])
          acc = acc + zi * zj * w
          ssum = ssum + w
          w = w * d
          tvec = tvec - 1
        return w, acc, ssum, tvec

      init = (jnp.ones((L,), jnp.float32),
              jnp.zeros((L,), jnp.float32),
              jnp.zeros((L,), jnp.float32),
              jnp.full((L,), T - 1, jnp.int32))
      res = lax.fori_loop(0, T // UNROLL, t_chunk, init)
      acc, ssum = res[1], res[2]

      o_v[...] = acc
      pltpu.sync_copy(o_v, num_hbm.at[b, pl.ds(pbase, L)])
      if b == 0:
        s_v[...] = ssum
        pltpu.sync_copy(s_v, s_hbm.at[pl.ds(pbase, L)])
    return carry0

  lax.fori_loop(0, GPW, group_body, None)


_sc_call = functools.partial(
    pl.kernel,
    mesh=plsc.VectorSubcoreMesh(core_axis_name="c", subcore_axis_name="s"),
    compiler_params=pltpu.CompilerParams(
        use_tc_tiling_on_sc=False, needs_layout_passes=False),
    out_type=[jax.ShapeDtypeStruct((B, N), jnp.float32),
              jax.ShapeDtypeStruct((N,), jnp.float32)],
    scratch_types=[
        pltpu.VMEM((L,), jnp.int32),
        pltpu.VMEM((L,), jnp.int32),
        pltpu.VMEM((L,), jnp.int32),
        pltpu.VMEM((L,), jnp.float32),
        pltpu.VMEM((L, T), jnp.float32),
        pltpu.VMEM((L, T), jnp.float32),
        pltpu.VMEM((L,), jnp.float32),
        pltpu.VMEM((L,), jnp.float32),
        pltpu.SemaphoreType.DMA,
    ],
)(_sc_body)


@jax.jit
def kernel(z_hist, decay_rates, idx_i, idx_j):
  zt = jnp.transpose(z_hist, (0, 2, 1)).reshape(B * D, T)
  r = jax.nn.softplus(decay_rates)
  num, s = _sc_call(zt, r, idx_i.astype(jnp.int32), idx_j.astype(jnp.int32))
  return num / jnp.sqrt(s + EPS)[None, :]


# speedup vs baseline: 7.3104x; 7.3104x over previous
import functools

import jax
import jax.numpy as jnp
from jax import lax
from jax.experimental import pallas as pl
from jax.experimental.pallas import tpu as pltpu
from jax.experimental.pallas import tpu_sc as plsc

D = 2048
T = 2048
B = 2
N = 8192
EPS = 1e-8

NC = 2
NS = 16
NW = NC * NS
L = 16

GROUPS = N // L
GPW = GROUPS // NW
PPW = GPW * L
NSEG = 16
TSEG = T // NSEG
UNROLL = 8
CUT = 23.03


def _sc_body(zt_hbm, r_hbm, ii_hbm, jj_hbm, num_hbm, s_hbm,
             ii_all, jj_all, r_all, ridx_i, ridx_j,
             rows_i, rows_j, num0_st, num1_st, s_st, sem):
  wid = lax.axis_index("s") * NC + lax.axis_index("c")
  lanes = lax.iota(jnp.int32, L)
  base = wid * PPW
  pltpu.sync_copy(ii_hbm.at[pl.ds(base, PPW)], ii_all)
  pltpu.sync_copy(jj_hbm.at[pl.ds(base, PPW)], jj_all)
  pltpu.sync_copy(r_hbm.at[pl.ds(base, PPW)], r_all)

  def group_body(gl, carry0):
    ii = ii_all[pl.ds(gl * L, L)]
    jj = jj_all[pl.ds(gl * L, L)]
    r_v = r_all[pl.ds(gl * L, L)]
    d = jnp.exp(-r_v)
    r_min = jnp.min(r_v)
    seg_start = lanes.astype(jnp.float32) * (r_min * float(TSEG))
    n_segs = jnp.sum((seg_start < CUT).astype(jnp.int32))

    for b in range(B):
      row_i = (ii + b * D) * NSEG
      row_j = (jj + b * D) * NSEG

      def seg_body(s, seg_carry):
        w, acc, ssum = seg_carry
        ridx_i[...] = row_i + (NSEG - 1 - s)
        ridx_j[...] = row_j + (NSEG - 1 - s)
        cp_i = pltpu.async_copy(zt_hbm.at[ridx_i], rows_i, sem)
        cp_j = pltpu.async_copy(zt_hbm.at[ridx_j], rows_j, sem)
        cp_i.wait()
        cp_j.wait()

        def t_chunk(c, ch_carry):
          w, acc, ssum, tvec = ch_carry
          for _ in range(UNROLL):
            zi = plsc.load_gather(rows_i, [lanes, tvec])
            zj = plsc.load_gather(rows_j, [lanes, tvec])
            acc = acc + zi * zj * w
            ssum = ssum + w
            w = w * d
            tvec = tvec - 1
          return w, acc, ssum, tvec

        init = (w, acc, ssum, jnp.full((L,), TSEG - 1, jnp.int32))
        res = lax.fori_loop(0, TSEG // UNROLL, t_chunk, init)
        return res[0], res[1], res[2]

      init = (jnp.ones((L,), jnp.float32),
              jnp.zeros((L,), jnp.float32),
              jnp.zeros((L,), jnp.float32))
      _, acc, ssum = lax.fori_loop(0, n_segs, seg_body, init)

      if b == 0:
        num0_st[pl.ds(gl * L, L)] = acc
        s_st[pl.ds(gl * L, L)] = ssum
      else:
        num1_st[pl.ds(gl * L, L)] = acc
    return carry0

  lax.fori_loop(0, GPW, group_body, None)

  pltpu.sync_copy(num0_st, num_hbm.at[0, pl.ds(base, PPW)])
  pltpu.sync_copy(num1_st, num_hbm.at[1, pl.ds(base, PPW)])
  pltpu.sync_copy(s_st, s_hbm.at[pl.ds(base, PPW)])


_sc_call = functools.partial(
    pl.kernel,
    mesh=plsc.VectorSubcoreMesh(core_axis_name="c", subcore_axis_name="s"),
    compiler_params=pltpu.CompilerParams(
        use_tc_tiling_on_sc=False, needs_layout_passes=False),
    out_type=[jax.ShapeDtypeStruct((B, N), jnp.float32),
              jax.ShapeDtypeStruct((N,), jnp.float32)],
    scratch_types=[
        pltpu.VMEM((PPW,), jnp.int32),
        pltpu.VMEM((PPW,), jnp.int32),
        pltpu.VMEM((PPW,), jnp.float32),
        pltpu.VMEM((L,), jnp.int32),
        pltpu.VMEM((L,), jnp.int32),
        pltpu.VMEM((L, TSEG), jnp.float32),
        pltpu.VMEM((L, TSEG), jnp.float32),
        pltpu.VMEM((PPW,), jnp.float32),
        pltpu.VMEM((PPW,), jnp.float32),
        pltpu.VMEM((PPW,), jnp.float32),
        pltpu.SemaphoreType.DMA,
    ],
)(_sc_body)


@jax.jit
def kernel(z_hist, decay_rates, idx_i, idx_j):
  zt = jnp.transpose(z_hist, (0, 2, 1)).reshape(B * D * NSEG, TSEG)
  r = jax.nn.softplus(decay_rates)
  num, s = _sc_call(zt, r, idx_i.astype(jnp.int32), idx_j.astype(jnp.int32))
  return num / jnp.sqrt(s + EPS)[None, :]


# speedup vs baseline: 9.3996x; 1.2858x over previous
import functools

import jax
import jax.numpy as jnp
from jax import lax
from jax.experimental import pallas as pl
from jax.experimental.pallas import tpu as pltpu
from jax.experimental.pallas import tpu_sc as plsc

D = 2048
T = 2048
B = 2
N = 8192
EPS = 1e-8

NC = 2
NS = 16
NW = NC * NS
L = 16

GROUPS = N // L
GPW = GROUPS // NW
PPW = GPW * L
NSEG = 32
TSEG = T // NSEG
UNROLL = 8
CUT = 23.03


def _sc_body(zt_hbm, r_hbm, ii_hbm, jj_hbm, num_hbm, s_hbm,
             ii_all, jj_all, r_all, ridx,
             rows, num0_st, num1_st, s_st, sem):
  wid = lax.axis_index("s") * NC + lax.axis_index("c")
  lanes = lax.iota(jnp.int32, L)
  base = wid * PPW
  pltpu.sync_copy(ii_hbm.at[pl.ds(base, PPW)], ii_all)
  pltpu.sync_copy(jj_hbm.at[pl.ds(base, PPW)], jj_all)
  pltpu.sync_copy(r_hbm.at[pl.ds(base, PPW)], r_all)

  def group_body(gl, carry0):
    ii = ii_all[pl.ds(gl * L, L)]
    jj = jj_all[pl.ds(gl * L, L)]
    r_v = r_all[pl.ds(gl * L, L)]
    d = jnp.exp(-r_v)
    r_min = jnp.min(r_v)
    lanes_f = lanes.astype(jnp.float32)
    step = r_min * float(TSEG)
    n_segs = jnp.sum((lanes_f * step < CUT).astype(jnp.int32))
    n_segs = n_segs + jnp.sum(((lanes_f + float(L)) * step < CUT).astype(jnp.int32))

    for b in range(B):
      row_i = (ii + b * D) * NSEG
      row_j = (jj + b * D) * NSEG

      def seg_body(s, seg_carry):
        w, acc, ssum = seg_carry
        ridx[pl.ds(0, L)] = row_i + (NSEG - 1 - s)
        ridx[pl.ds(L, L)] = row_j + (NSEG - 1 - s)
        pltpu.async_copy(zt_hbm.at[ridx], rows, sem).wait()

        def t_chunk(c, ch_carry):
          w, acc, ssum, tvec = ch_carry
          for _ in range(UNROLL):
            zi = plsc.load_gather(rows, [lanes, tvec])
            zj = plsc.load_gather(rows, [lanes + L, tvec])
            acc = acc + zi * zj * w
            ssum = ssum + w
            w = w * d
            tvec = tvec - 1
          return w, acc, ssum, tvec

        init = (w, acc, ssum, jnp.full((L,), TSEG - 1, jnp.int32))
        res = lax.fori_loop(0, TSEG // UNROLL, t_chunk, init)
        return res[0], res[1], res[2]

      init = (jnp.ones((L,), jnp.float32),
              jnp.zeros((L,), jnp.float32),
              jnp.zeros((L,), jnp.float32))
      _, acc, ssum = lax.fori_loop(0, n_segs, seg_body, init)

      if b == 0:
        num0_st[pl.ds(gl * L, L)] = acc
        s_st[pl.ds(gl * L, L)] = ssum
      else:
        num1_st[pl.ds(gl * L, L)] = acc
    return carry0

  lax.fori_loop(0, GPW, group_body, None)

  pltpu.sync_copy(num0_st, num_hbm.at[0, pl.ds(base, PPW)])
  pltpu.sync_copy(num1_st, num_hbm.at[1, pl.ds(base, PPW)])
  pltpu.sync_copy(s_st, s_hbm.at[pl.ds(base, PPW)])


_sc_call = functools.partial(
    pl.kernel,
    mesh=plsc.VectorSubcoreMesh(core_axis_name="c", subcore_axis_name="s"),
    compiler_params=pltpu.CompilerParams(
        use_tc_tiling_on_sc=False, needs_layout_passes=False),
    out_type=[jax.ShapeDtypeStruct((B, N), jnp.float32),
              jax.ShapeDtypeStruct((N,), jnp.float32)],
    scratch_types=[
        pltpu.VMEM((PPW,), jnp.int32),
        pltpu.VMEM((PPW,), jnp.int32),
        pltpu.VMEM((PPW,), jnp.float32),
        pltpu.VMEM((2 * L,), jnp.int32),
        pltpu.VMEM((2 * L, TSEG), jnp.float32),
        pltpu.VMEM((PPW,), jnp.float32),
        pltpu.VMEM((PPW,), jnp.float32),
        pltpu.VMEM((PPW,), jnp.float32),
        pltpu.SemaphoreType.DMA,
    ],
)(_sc_body)


@jax.jit
def kernel(z_hist, decay_rates, idx_i, idx_j):
  zt = jnp.transpose(z_hist, (0, 2, 1)).reshape(B * D * NSEG, TSEG)
  r = jax.nn.softplus(decay_rates)
  num, s = _sc_call(zt, r, idx_i.astype(jnp.int32), idx_j.astype(jnp.int32))
  return num / jnp.sqrt(s + EPS)[None, :]
